# Initial kernel scaffold; baseline (speedup 1.0000x reference)
import jax
import jax.numpy as jnp
from jax.experimental import pallas as pl


def kernel(model_out, tgt):
    raise NotImplementedError("write your pallas kernel here")


# speedup vs baseline: 1.2505x; 1.2505x over previous
import jax
import jax.numpy as jnp
from jax.experimental import pallas as pl
from jax.experimental.pallas import tpu as pltpu

_ALPHA = 0.05


def _row_loss_kernel(x_ref, t_ref, out_ref, acc_ref):
    i = pl.program_id(0)
    nb = pl.num_programs(0)
    x = x_ref[...]
    R, V = x.shape
    t = t_ref[0, 0, :].reshape(R, 1)
    bm = jnp.max(x, axis=1, keepdims=True)
    se = jnp.sum(jnp.exp(x - bm), axis=1, keepdims=True)
    lse = bm + jnp.log(se)
    sx = jnp.sum(x, axis=1, keepdims=True)
    cols = jax.lax.broadcasted_iota(jnp.int32, (R, V), 1)
    xt = jnp.sum(jnp.where(cols == t, x, 0.0), axis=1, keepdims=True)
    base = (_ALPHA / V) * sx - lse + (1.0 - _ALPHA) * xt
    contrib = jnp.sum(jnp.where(t != 0, base, 0.0))
    cnt = jnp.sum(jnp.where(t != 0, 1.0, 0.0))

    @pl.when(i == 0)
    def _init():
        acc_ref[0] = 0.0
        acc_ref[1] = 0.0

    acc_ref[0] += contrib
    acc_ref[1] += cnt

    @pl.when(i == nb - 1)
    def _fin():
        out_ref[0, 0] = -acc_ref[0] / acc_ref[1]


def kernel(model_out, tgt):
    S, B, V = model_out.shape
    tgt = tgt.astype(jnp.int32)
    t_shift = jnp.roll(tgt, -1, axis=1).at[:, -1].set(0)
    t_flat = jnp.transpose(t_shift).reshape(-1)
    x = model_out.reshape(S * B, V)
    R = 8
    nb = (S * B) // R
    t3 = t_flat.reshape(nb, 1, R)
    out = pl.pallas_call(
        _row_loss_kernel,
        grid=(nb,),
        in_specs=[
            pl.BlockSpec((R, V), lambda i: (i, 0)),
            pl.BlockSpec((1, 1, R), lambda i: (i, 0, 0)),
        ],
        out_specs=pl.BlockSpec(memory_space=pltpu.SMEM),
        out_shape=jax.ShapeDtypeStruct((1, 1), jnp.float32),
        scratch_shapes=[pltpu.SMEM((2,), jnp.float32)],
        compiler_params=pltpu.CompilerParams(dimension_semantics=("arbitrary",)),
    )(x, t3)
    return out[0, 0]


# speedup vs baseline: 1.3340x; 1.0667x over previous
import jax
import jax.numpy as jnp
from jax.experimental import pallas as pl
from jax.experimental.pallas import tpu as pltpu

_ALPHA = 0.05
_NACC = 4
_LANES = 128


def _row_loss_kernel(x_ref, t_ref, out_ref, acc_ref):
    i = pl.program_id(0)
    nb = pl.num_programs(0)
    R = x_ref.shape[1]
    V = x_ref.shape[2]
    nc = V // _LANES
    tail_w = V - nc * _LANES

    t = t_ref[0, 0, :].reshape(R, 1)
    ti = jnp.broadcast_to(t, (R, _LANES))
    iota = jax.lax.broadcasted_iota(jnp.int32, (R, _LANES), 1)

    macc = []
    sacc = []
    gacc = []
    for a in range(_NACC):
        c = x_ref[0, :, a * _LANES:(a + 1) * _LANES]
        macc.append(c)
        sacc.append(c)
        gacc.append(jnp.where(iota + (a * _LANES) == ti, c, 0.0))
    for k in range(_NACC, nc):
        c = x_ref[0, :, k * _LANES:(k + 1) * _LANES]
        a = k % _NACC
        macc[a] = jnp.maximum(macc[a], c)
        sacc[a] = sacc[a] + c
        gacc[a] = gacc[a] + jnp.where(iota + (k * _LANES) == ti, c, 0.0)
    m128 = jnp.maximum(jnp.maximum(macc[0], macc[1]), jnp.maximum(macc[2], macc[3]))
    s128 = (sacc[0] + sacc[1]) + (sacc[2] + sacc[3])
    g128 = (gacc[0] + gacc[1]) + (gacc[2] + gacc[3])
    tail = x_ref[0, :, nc * _LANES:V]
    m = jnp.maximum(jnp.max(m128, axis=1, keepdims=True),
                    jnp.max(tail, axis=1, keepdims=True))
    sx = jnp.sum(s128, axis=1, keepdims=True) + jnp.sum(tail, axis=1, keepdims=True)
    iota_t = jax.lax.broadcasted_iota(jnp.int32, (R, tail_w), 1)
    g_tail = jnp.where(iota_t + nc * _LANES == t, tail, 0.0)
    xt = jnp.sum(g128, axis=1, keepdims=True) + jnp.sum(g_tail, axis=1, keepdims=True)

    mb = jnp.broadcast_to(m, (R, _LANES))
    eacc = []
    for a in range(_NACC):
        c = x_ref[0, :, a * _LANES:(a + 1) * _LANES]
        eacc.append(jnp.exp(c - mb))
    for k in range(_NACC, nc):
        c = x_ref[0, :, k * _LANES:(k + 1) * _LANES]
        a = k % _NACC
        eacc[a] = eacc[a] + jnp.exp(c - mb)
    e128 = (eacc[0] + eacc[1]) + (eacc[2] + eacc[3])
    se = jnp.sum(e128, axis=1, keepdims=True) + \
        jnp.sum(jnp.exp(tail - m), axis=1, keepdims=True)
    lse = m + jnp.log(se)

    base = (_ALPHA / V) * sx - lse + (1.0 - _ALPHA) * xt
    contrib = jnp.sum(jnp.where(t != 0, base, 0.0))
    cnt = jnp.sum(jnp.where(t != 0, 1.0, 0.0))

    @pl.when(i == 0)
    def _init():
        acc_ref[0] = 0.0
        acc_ref[1] = 0.0

    acc_ref[0] += contrib
    acc_ref[1] += cnt

    @pl.when(i == nb - 1)
    def _fin():
        out_ref[0, 0] = -acc_ref[0] / acc_ref[1]


def kernel(model_out, tgt):
    S, B, V = model_out.shape
    tgt = tgt.astype(jnp.int32)
    t_shift = jnp.roll(tgt, -1, axis=1).at[:, -1].set(0)
    t_flat = jnp.transpose(t_shift).reshape(-1)
    R = B
    nb = S
    t3 = t_flat.reshape(nb, 1, R)
    out = pl.pallas_call(
        _row_loss_kernel,
        grid=(nb,),
        in_specs=[
            pl.BlockSpec((1, R, V), lambda i: (i, 0, 0)),
            pl.BlockSpec((1, 1, R), lambda i: (i, 0, 0)),
        ],
        out_specs=pl.BlockSpec(memory_space=pltpu.SMEM),
        out_shape=jax.ShapeDtypeStruct((1, 1), jnp.float32),
        scratch_shapes=[pltpu.SMEM((2,), jnp.float32)],
        compiler_params=pltpu.CompilerParams(dimension_semantics=("arbitrary",)),
    )(model_out, t3)
    return out[0, 0]


# speedup vs baseline: 1.6279x; 1.2204x over previous
import jax
import jax.numpy as jnp
from jax.experimental import pallas as pl
from jax.experimental.pallas import tpu as pltpu

_ALPHA = 0.05
_NACC = 4
_LANES = 128


def _row_loss_kernel(x_ref, t_ref, out_ref, acc_ref):
    i = pl.program_id(0)
    nb = pl.num_programs(0)
    G = x_ref.shape[0]
    R = x_ref.shape[1]
    V = x_ref.shape[2]
    nc = V // _LANES
    tail_w = V - nc * _LANES

    t = t_ref[...]
    ti = jnp.broadcast_to(t, (G, R, _LANES))
    iota = jax.lax.broadcasted_iota(jnp.int32, (G, R, _LANES), 2)

    macc = []
    sacc = []
    gacc = []
    for a in range(_NACC):
        c = x_ref[:, :, a * _LANES:(a + 1) * _LANES]
        macc.append(c)
        sacc.append(c)
        gacc.append(jnp.where(iota + (a * _LANES) == ti, c, 0.0))
    for k in range(_NACC, nc):
        c = x_ref[:, :, k * _LANES:(k + 1) * _LANES]
        a = k % _NACC
        macc[a] = jnp.maximum(macc[a], c)
        sacc[a] = sacc[a] + c
        gacc[a] = gacc[a] + jnp.where(iota + (k * _LANES) == ti, c, 0.0)
    m128 = jnp.maximum(jnp.maximum(macc[0], macc[1]), jnp.maximum(macc[2], macc[3]))
    s128 = (sacc[0] + sacc[1]) + (sacc[2] + sacc[3])
    g128 = (gacc[0] + gacc[1]) + (gacc[2] + gacc[3])
    tail = x_ref[:, :, nc * _LANES:V]
    m = jnp.maximum(jnp.max(m128, axis=2, keepdims=True),
                    jnp.max(tail, axis=2, keepdims=True))
    sx = jnp.sum(s128, axis=2, keepdims=True) + jnp.sum(tail, axis=2, keepdims=True)
    iota_t = jax.lax.broadcasted_iota(jnp.int32, (G, R, tail_w), 2)
    g_tail = jnp.where(iota_t + nc * _LANES == t, tail, 0.0)
    xt = jnp.sum(g128, axis=2, keepdims=True) + jnp.sum(g_tail, axis=2, keepdims=True)

    mb = jnp.broadcast_to(m, (G, R, _LANES))
    eacc = []
    for a in range(_NACC):
        c = x_ref[:, :, a * _LANES:(a + 1) * _LANES]
        eacc.append(jnp.exp(c - mb))
    for k in range(_NACC, nc):
        c = x_ref[:, :, k * _LANES:(k + 1) * _LANES]
        a = k % _NACC
        eacc[a] = eacc[a] + jnp.exp(c - mb)
    e128 = (eacc[0] + eacc[1]) + (eacc[2] + eacc[3])
    se = jnp.sum(e128, axis=2, keepdims=True) + \
        jnp.sum(jnp.exp(tail - m), axis=2, keepdims=True)
    lse = m + jnp.log(se)

    base = (_ALPHA / V) * sx - lse + (1.0 - _ALPHA) * xt
    contrib = jnp.sum(jnp.where(t != 0, base, 0.0))
    cnt = jnp.sum(jnp.where(t != 0, 1.0, 0.0))

    @pl.when(i == 0)
    def _init():
        acc_ref[0] = 0.0
        acc_ref[1] = 0.0

    acc_ref[0] += contrib
    acc_ref[1] += cnt

    @pl.when(i == nb - 1)
    def _fin():
        out_ref[0, 0] = -acc_ref[0] / acc_ref[1]


def kernel(model_out, tgt):
    S, B, V = model_out.shape
    tgt = tgt.astype(jnp.int32)
    t_shift = jnp.roll(tgt, -1, axis=1).at[:, -1].set(0)
    t_flat = jnp.transpose(t_shift).reshape(-1)
    G = 4
    nb = S // G
    t3 = t_flat.reshape(S, B, 1)
    out = pl.pallas_call(
        _row_loss_kernel,
        grid=(nb,),
        in_specs=[
            pl.BlockSpec((G, B, V), lambda i: (i, 0, 0)),
            pl.BlockSpec((G, B, 1), lambda i: (i, 0, 0)),
        ],
        out_specs=pl.BlockSpec(memory_space=pltpu.SMEM),
        out_shape=jax.ShapeDtypeStruct((1, 1), jnp.float32),
        scratch_shapes=[pltpu.SMEM((2,), jnp.float32)],
        compiler_params=pltpu.CompilerParams(dimension_semantics=("arbitrary",)),
    )(model_out, t3)
    return out[0, 0]


# speedup vs baseline: 1.6685x; 1.0249x over previous
import jax
import jax.numpy as jnp
from jax.experimental import pallas as pl
from jax.experimental.pallas import tpu as pltpu

_ALPHA = 0.05
_NACC = 4
_LANES = 128


def _row_loss_kernel(x_ref, t_ref, out_ref, acc_ref):
    i = pl.program_id(0)
    nb = pl.num_programs(0)
    G = x_ref.shape[0]
    R = x_ref.shape[1]
    V = x_ref.shape[2]
    nc = V // _LANES
    tail_w = V - nc * _LANES

    t = t_ref[...]
    ti = jnp.broadcast_to(t, (G, R, _LANES))
    iota = jax.lax.broadcasted_iota(jnp.int32, (G, R, _LANES), 2)

    macc = []
    sacc = []
    gacc = []
    for a in range(_NACC):
        c = x_ref[:, :, a * _LANES:(a + 1) * _LANES]
        macc.append(c)
        sacc.append(c)
        gacc.append(jnp.where(iota + (a * _LANES) == ti, c, 0.0))
    for k in range(_NACC, nc):
        c = x_ref[:, :, k * _LANES:(k + 1) * _LANES]
        a = k % _NACC
        macc[a] = jnp.maximum(macc[a], c)
        sacc[a] = sacc[a] + c
        gacc[a] = gacc[a] + jnp.where(iota + (k * _LANES) == ti, c, 0.0)
    m128 = jnp.maximum(jnp.maximum(macc[0], macc[1]), jnp.maximum(macc[2], macc[3]))
    s128 = (sacc[0] + sacc[1]) + (sacc[2] + sacc[3])
    g128 = (gacc[0] + gacc[1]) + (gacc[2] + gacc[3])
    tail = x_ref[:, :, nc * _LANES:V]
    m = jnp.maximum(jnp.max(m128, axis=2, keepdims=True),
                    jnp.max(tail, axis=2, keepdims=True))
    sx = jnp.sum(s128, axis=2, keepdims=True) + jnp.sum(tail, axis=2, keepdims=True)
    iota_t = jax.lax.broadcasted_iota(jnp.int32, (G, R, tail_w), 2)
    g_tail = jnp.where(iota_t + nc * _LANES == t, tail, 0.0)
    xt = jnp.sum(g128, axis=2, keepdims=True) + jnp.sum(g_tail, axis=2, keepdims=True)

    mb = jnp.broadcast_to(m, (G, R, _LANES))
    eacc = []
    for a in range(_NACC):
        c = x_ref[:, :, a * _LANES:(a + 1) * _LANES]
        eacc.append(jnp.exp(c - mb))
    for k in range(_NACC, nc):
        c = x_ref[:, :, k * _LANES:(k + 1) * _LANES]
        a = k % _NACC
        eacc[a] = eacc[a] + jnp.exp(c - mb)
    e128 = (eacc[0] + eacc[1]) + (eacc[2] + eacc[3])
    se = jnp.sum(e128, axis=2, keepdims=True) + \
        jnp.sum(jnp.exp(tail - m), axis=2, keepdims=True)
    lse = m + jnp.log(se)

    base = (_ALPHA / V) * sx - lse + (1.0 - _ALPHA) * xt
    contrib = jnp.sum(jnp.where(t != 0, base, 0.0))
    cnt = jnp.sum(jnp.where(t != 0, 1.0, 0.0))

    @pl.when(i == 0)
    def _init():
        acc_ref[0] = 0.0
        acc_ref[1] = 0.0

    acc_ref[0] += contrib
    acc_ref[1] += cnt

    @pl.when(i == nb - 1)
    def _fin():
        out_ref[0, 0] = -acc_ref[0] / acc_ref[1]


def kernel(model_out, tgt):
    S, B, V = model_out.shape
    tgt = tgt.astype(jnp.int32)
    t_shift = jnp.roll(tgt, -1, axis=1).at[:, -1].set(0)
    t_flat = jnp.transpose(t_shift).reshape(-1)
    G = 8
    nb = S // G
    t3 = t_flat.reshape(S, B, 1)
    out = pl.pallas_call(
        _row_loss_kernel,
        grid=(nb,),
        in_specs=[
            pl.BlockSpec((G, B, V), lambda i: (i, 0, 0)),
            pl.BlockSpec((G, B, 1), lambda i: (i, 0, 0)),
        ],
        out_specs=pl.BlockSpec(memory_space=pltpu.SMEM),
        out_shape=jax.ShapeDtypeStruct((1, 1), jnp.float32),
        scratch_shapes=[pltpu.SMEM((2,), jnp.float32)],
        compiler_params=pltpu.CompilerParams(dimension_semantics=("arbitrary",)),
    )(model_out, t3)
    return out[0, 0]


# speedup vs baseline: 5.2328x; 3.1363x over previous
import jax
import jax.numpy as jnp
from jax.experimental import pallas as pl
from jax.experimental.pallas import tpu as pltpu

_ALPHA = 0.05
_W = 1733
_U = 173


def _loss_kernel(x_ref, t_ref, out_ref, m_run, se_run, sx_run, gx_run):
    i = pl.program_id(0)
    nb = pl.num_programs(0)
    V = nb * _W
    t = t_ref[...]
    t_rel = t - i * _W
    neg_inf = jnp.full(t.shape, -jnp.inf, jnp.float32)
    zero = jnp.zeros(t.shape, jnp.float32)

    @pl.when(i == 0)
    def _init():
        m_run[...] = neg_inf
        se_run[...] = zero
        sx_run[...] = zero
        gx_run[...] = zero

    def p1(j, carry):
        m0, m1, s0, s1, g0, g1 = carry
        tg = t_rel - j * _U
        for u in range(_U):
            c = x_ref[j * _U + u]
            sel = jnp.where(tg == u, c, 0.0)
            if u % 2 == 0:
                m0 = jnp.maximum(m0, c)
                s0 = s0 + c
                g0 = g0 + sel
            else:
                m1 = jnp.maximum(m1, c)
                s1 = s1 + c
                g1 = g1 + sel
        return m0, m1, s0, s1, g0, g1

    m0, m1, s0, s1, g0, g1 = jax.lax.fori_loop(
        0, _W // _U, p1, (neg_inf, neg_inf, zero, zero, zero, zero))
    for k in range((_W // _U) * _U, _W):
        c = x_ref[k]
        m0 = jnp.maximum(m0, c)
        s0 = s0 + c
        g0 = g0 + jnp.where(t_rel == k, c, 0.0)
    bm = jnp.maximum(m0, m1)
    m_old = m_run[...]
    m_new = jnp.maximum(m_old, bm)

    def p2(j, carry):
        e0, e1 = carry
        for u in range(_U):
            c = x_ref[j * _U + u]
            e = jnp.exp(c - m_new)
            if u % 2 == 0:
                e0 = e0 + e
            else:
                e1 = e1 + e
        return e0, e1

    e0, e1 = jax.lax.fori_loop(0, _W // _U, p2, (zero, zero))
    for k in range((_W // _U) * _U, _W):
        e0 = e0 + jnp.exp(x_ref[k] - m_new)

    m_run[...] = m_new
    se_run[...] = se_run[...] * jnp.exp(m_old - m_new) + (e0 + e1)
    sx_run[...] = sx_run[...] + (s0 + s1)
    gx_run[...] = gx_run[...] + (g0 + g1)

    @pl.when(i == nb - 1)
    def _fin():
        lse = m_run[...] + jnp.log(se_run[...])
        base = (_ALPHA / V) * sx_run[...] - lse + (1.0 - _ALPHA) * gx_run[...]
        mask = t != 0
        contrib = jnp.sum(jnp.where(mask, base, 0.0))
        cnt = jnp.sum(jnp.where(mask, 1.0, 0.0))
        out_ref[0, 0] = -contrib / cnt


def kernel(model_out, tgt):
    S, B, V = model_out.shape
    xt = jnp.transpose(model_out, (2, 1, 0))
    tgt = tgt.astype(jnp.int32)
    t_shift = jnp.roll(tgt, -1, axis=1).at[:, -1].set(0)
    nb = V // _W
    out = pl.pallas_call(
        _loss_kernel,
        grid=(nb,),
        in_specs=[
            pl.BlockSpec((_W, B, S), lambda i: (i, 0, 0)),
            pl.BlockSpec((B, S), lambda i: (0, 0)),
        ],
        out_specs=pl.BlockSpec(memory_space=pltpu.SMEM),
        out_shape=jax.ShapeDtypeStruct((1, 1), jnp.float32),
        scratch_shapes=[pltpu.VMEM((B, S), jnp.float32) for _ in range(4)],
        compiler_params=pltpu.CompilerParams(dimension_semantics=("arbitrary",)),
    )(xt, t_shift)
    return out[0, 0]
